# baseline (speedup 1.0000x reference)
import math

import jax
import jax.numpy as jnp
from jax.experimental import pallas as pl
from jax.experimental.pallas import tpu as pltpu

_PACK = 32


def _mlp_packed_kernel(x_ref, w1_ref, b1_ref, w2_ref, b2_ref, w3_ref, b3_ref,
                       o_ref):
    x = x_ref[...]
    h1 = jax.nn.sigmoid(
        jnp.dot(x, w1_ref[...], preferred_element_type=jnp.float32)
        + b1_ref[...])
    h2 = jax.nn.sigmoid(
        jnp.dot(h1, w2_ref[...], preferred_element_type=jnp.float32)
        + b2_ref[...])
    o_ref[...] = jax.nn.sigmoid(
        jnp.dot(h2, w3_ref[...], preferred_element_type=jnp.float32)
        + b3_ref[...])


@jax.jit
def kernel(x, w1, b1, w2, b2, w3, b3):
    B, F = x.shape
    f32 = jnp.float32
    x = x.astype(f32)

    pad = (-B) % _PACK
    if pad:
        x = jnp.pad(x, ((0, pad), (0, 0)))
    rows = (B + pad) // _PACK
    x2 = x.reshape(rows, _PACK * F)

    eye = jnp.eye(_PACK, dtype=f32)
    w1big = jnp.kron(eye, w1.astype(f32).T)
    w2big = jnp.kron(eye, w2.astype(f32).T)
    w3d = (w3[1:2, :] - w3[0:1, :]).astype(f32)
    w3s = jnp.concatenate([-w3d.T, w3d.T], axis=1)
    w3big = jnp.kron(eye, w3s)
    b1big = jnp.tile(b1.astype(f32).reshape(1, -1), (1, _PACK))
    b2big = jnp.tile(b2.astype(f32).reshape(1, -1), (1, _PACK))
    b3d = (b3[1:2, :] - b3[0:1, :]).astype(f32).reshape(1, 1)
    b3big = jnp.tile(jnp.concatenate([-b3d, b3d], axis=1), (1, _PACK))

    br = math.gcd(rows, 1024)
    grid = rows // br
    full = lambda shape: pl.BlockSpec(shape, lambda i: (0, 0))

    out2 = pl.pallas_call(
        _mlp_packed_kernel,
        out_shape=jax.ShapeDtypeStruct((rows, 2 * _PACK), f32),
        grid=(grid,),
        in_specs=[
            pl.BlockSpec((br, F * _PACK), lambda i: (i, 0)),
            full((F * _PACK, F * _PACK)), full((1, F * _PACK)),
            full((F * _PACK, 8 * _PACK)), full((1, 8 * _PACK)),
            full((8 * _PACK, 2 * _PACK)), full((1, 2 * _PACK)),
        ],
        out_specs=pl.BlockSpec((br, 2 * _PACK), lambda i: (i, 0)),
        compiler_params=pltpu.CompilerParams(
            dimension_semantics=("parallel",)),
    )(x2, w1big, b1big, w2big, b2big, w3big, b3big)

    out = out2.reshape(-1, 2)
    if pad:
        out = out[:B]
    return out


# speedup vs baseline: 1.5471x; 1.5471x over previous
import math

import jax
import jax.numpy as jnp
from jax.experimental import pallas as pl
from jax.experimental.pallas import tpu as pltpu


def _mlp_kernel(x_ref, w1_ref, b1_ref, w2_ref, b2_ref, w3_ref, b3_ref, o_ref):
    x = x_ref[...]
    h1 = jax.nn.sigmoid(
        jax.lax.dot_general(w1_ref[...], x, (((1,), (1,)), ((), ())),
                            preferred_element_type=jnp.float32)
        + b1_ref[...])
    h2 = jax.nn.sigmoid(
        jnp.dot(w2_ref[...], h1, preferred_element_type=jnp.float32)
        + b2_ref[...])
    w3 = w3_ref[...]
    b3 = b3_ref[...]
    d = (jnp.dot(w3[1:2, :] - w3[0:1, :], h2,
                 preferred_element_type=jnp.float32)
         + (b3[1:2, :] - b3[0:1, :]))
    p = jax.nn.sigmoid(d)
    ct = jnp.concatenate([1.0 - p, p], axis=0)
    o_ref[...] = jax.lax.transpose(ct, (1, 0))


@jax.jit
def kernel(x, w1, b1, w2, b2, w3, b3):
    B, F = x.shape
    f32 = jnp.float32
    x = x.astype(f32)

    tb = math.gcd(B, 8192)
    grid = B // tb
    full = lambda shape: pl.BlockSpec(shape, lambda i: (0, 0))

    return pl.pallas_call(
        _mlp_kernel,
        out_shape=jax.ShapeDtypeStruct((B, 2), f32),
        grid=(grid,),
        in_specs=[
            pl.BlockSpec((tb, F), lambda i: (i, 0)),
            full((12, 12)), full((12, 1)),
            full((8, 12)), full((8, 1)),
            full((2, 8)), full((2, 1)),
        ],
        out_specs=pl.BlockSpec((tb, 2), lambda i: (i, 0)),
        compiler_params=pltpu.CompilerParams(
            dimension_semantics=("parallel",)),
    )(x, w1.astype(f32), b1.astype(f32), w2.astype(f32), b2.astype(f32),
      w3.astype(f32), b3.astype(f32))


# speedup vs baseline: 17.8169x; 11.5161x over previous
import math

import jax
import jax.numpy as jnp
from jax.experimental import pallas as pl
from jax.experimental.pallas import tpu as pltpu


def _mlp_kernel(x_ref, w1_ref, b1_ref, w2_ref, b2_ref, w3_ref, b3_ref, o_ref):
    x = x_ref[...]
    h1 = jax.nn.sigmoid(
        jnp.dot(w1_ref[...], x, preferred_element_type=jnp.float32)
        + b1_ref[...])
    h2 = jax.nn.sigmoid(
        jnp.dot(w2_ref[...], h1, preferred_element_type=jnp.float32)
        + b2_ref[...])
    w3 = w3_ref[...]
    b3 = b3_ref[...]
    d = (jnp.dot(w3[1:2, :] - w3[0:1, :], h2,
                 preferred_element_type=jnp.float32)
         + (b3[1:2, :] - b3[0:1, :]))
    p1 = jax.nn.sigmoid(d)
    o_ref[...] = jnp.concatenate([1.0 - p1, p1], axis=0)


@jax.jit
def kernel(x, w1, b1, w2, b2, w3, b3):
    B, F = x.shape
    f32 = jnp.float32
    xT = x.astype(f32).T

    tb = math.gcd(B, 16384)
    grid = B // tb
    full = lambda shape: pl.BlockSpec(shape, lambda i: (0, 0))

    out = pl.pallas_call(
        _mlp_kernel,
        out_shape=jax.ShapeDtypeStruct((2, B), f32),
        grid=(grid,),
        in_specs=[
            pl.BlockSpec((F, tb), lambda i: (0, i)),
            full((12, 12)), full((12, 1)),
            full((8, 12)), full((8, 1)),
            full((2, 8)), full((2, 1)),
        ],
        out_specs=pl.BlockSpec((2, tb), lambda i: (0, i)),
        compiler_params=pltpu.CompilerParams(
            dimension_semantics=("parallel",)),
    )(xT, w1.astype(f32), b1.astype(f32), w2.astype(f32), b2.astype(f32),
      w3.astype(f32), b3.astype(f32))

    return out.T


# speedup vs baseline: 25.4304x; 1.4273x over previous
import math

import jax
import jax.numpy as jnp
from jax.experimental import pallas as pl
from jax.experimental.pallas import tpu as pltpu


def _mlp_kernel(x_ref, w1_ref, b1_ref, w2_ref, b2_ref, w3_ref, b3_ref, o_ref):
    x = x_ref[...]
    h1 = jax.nn.sigmoid(
        jnp.dot(w1_ref[...], x, preferred_element_type=jnp.float32)
        + b1_ref[...])
    h2 = jax.nn.sigmoid(
        jnp.dot(w2_ref[...], h1, preferred_element_type=jnp.float32)
        + b2_ref[...])
    w3 = w3_ref[...]
    b3 = b3_ref[...]
    d = (jnp.dot(w3[1:2, :] - w3[0:1, :], h2,
                 preferred_element_type=jnp.float32)
         + (b3[1:2, :] - b3[0:1, :]))
    p1 = jax.nn.sigmoid(d)
    o_ref[...] = jnp.concatenate([1.0 - p1, p1], axis=0)


@jax.jit
def kernel(x, w1, b1, w2, b2, w3, b3):
    B, F = x.shape
    f32 = jnp.float32
    xT = x.astype(f32).T

    tb = math.gcd(B, 65536)
    grid = B // tb
    full = lambda shape: pl.BlockSpec(shape, lambda i: (0, 0))

    out = pl.pallas_call(
        _mlp_kernel,
        out_shape=jax.ShapeDtypeStruct((2, B), f32),
        grid=(grid,),
        in_specs=[
            pl.BlockSpec((F, tb), lambda i: (0, i)),
            full((12, 12)), full((12, 1)),
            full((8, 12)), full((8, 1)),
            full((2, 8)), full((2, 1)),
        ],
        out_specs=pl.BlockSpec((2, tb), lambda i: (0, i)),
        compiler_params=pltpu.CompilerParams(
            dimension_semantics=("parallel",)),
    )(xT, w1.astype(f32), b1.astype(f32), w2.astype(f32), b2.astype(f32),
      w3.astype(f32), b3.astype(f32))

    return out.T
